# baseline (speedup 1.0000x reference)
import functools

import jax
import jax.numpy as jnp
import numpy as np
from jax import lax
from jax.experimental import pallas as pl
from jax.experimental.pallas import tpu as pltpu
from jax.experimental.pallas import tpu_sc as plsc

B, S = 64, 512
N = B * S
D = 256
DW = D // 2
NQ = 4

_info = plsc.get_sparse_core_info()
NC, NS = _info.num_cores, _info.num_subcores
NW = NC * NS
B_PER_W = N // NW
CHUNK = 32
NBUF = 2
N_CHUNKS = B_PER_W // CHUNK
N_GROUPS = N_CHUNKS // NBUF

_mesh = plsc.VectorSubcoreMesh(core_axis_name="c", subcore_axis_name="s")

_e = np.arange(D)
_g, _r = _e // 32, _e % 32
_t = np.where(_r % 2 == 0, _r // 2, 16 + (_r - 1) // 2)
_PERM = np.asarray(_g * 32 + _t, dtype=np.int32)


@functools.partial(
    pl.kernel,
    mesh=_mesh,
    out_type=jax.ShapeDtypeStruct((N, NQ * D), jnp.float32),
    scratch_types=(
        [pltpu.VMEM((B_PER_W,), jnp.int32) for _ in range(NQ)]
        + [pltpu.VMEM((CHUNK, NQ * DW), jnp.int32) for _ in range(NBUF)]
        + [pltpu.VMEM((CHUNK, NQ * D), jnp.float32) for _ in range(NBUF)]
        + [pltpu.SemaphoreType.DMA for _ in range(2 * NBUF)]
    ),
)
def _gather_kernel(i0, i1, i2, i3, wx, wy, wh, ww, out, *scratch):
    idx_refs = scratch[:NQ]
    gbuf = scratch[NQ: NQ + NBUF]
    rows = scratch[NQ + NBUF: NQ + 2 * NBUF]
    sem_g = scratch[NQ + 2 * NBUF: NQ + 3 * NBUF]
    sem_s = scratch[NQ + 3 * NBUF:]

    tables = (wx, wy, wh, ww)
    idx_hbm = (i0, i1, i3, i2)

    wid = lax.axis_index("s") * NC + lax.axis_index("c")
    base0 = pl.multiple_of(wid * B_PER_W, B_PER_W)

    for q in range(NQ):
        pltpu.sync_copy(idx_hbm[q].at[pl.ds(base0, B_PER_W)], idx_refs[q])

    mask = jnp.full((16,), jnp.int32(-65536))

    def convert_row(b, r, _):
        for w in range(NQ * DW // 16):
            x = gbuf[b][r, pl.ds(w * 16, 16)]
            lo = lax.bitcast_convert_type(lax.shift_left(x, 16), jnp.float32)
            hi = lax.bitcast_convert_type(lax.bitwise_and(x, mask),
                                          jnp.float32)
            rows[b][r, pl.ds(w * 32, 16)] = lo
            rows[b][r, pl.ds(w * 32 + 16, 16)] = hi
        return _

    def group(g, carry):
        goff = pl.multiple_of(g * (NBUF * CHUNK), NBUF * CHUNK)
        for b in range(NBUF):
            off = goff + b * CHUNK

            @pl.when(g > 0)
            def _drain_prev_stores():
                pltpu.make_async_copy(
                    rows[b], out.at[pl.ds(base0, CHUNK)], sem_s[b]).wait()

            for q in range(NQ):
                pltpu.async_copy(
                    tables[q].at[idx_refs[q].at[pl.ds(off, CHUNK)]],
                    gbuf[b].at[:, pl.ds(q * DW, DW)], sem_g[b])
        for b in range(NBUF):
            base = base0 + goff + b * CHUNK
            for q in range(NQ):
                pltpu.make_async_copy(
                    tables[q].at[idx_refs[q].at[pl.ds(0, CHUNK)]],
                    gbuf[b].at[:, pl.ds(q * DW, DW)], sem_g[b]).wait()
            lax.fori_loop(0, CHUNK, functools.partial(convert_row, b), 0)
            pltpu.async_copy(rows[b], out.at[pl.ds(base, CHUNK)], sem_s[b])
        return carry

    lax.fori_loop(0, N_GROUPS, group, 0)

    for b in range(NBUF):
        pltpu.make_async_copy(
            rows[b], out.at[pl.ds(base0, CHUNK)], sem_s[b]).wait()


def _prep_table(w):
    wbf = w[:, _PERM].astype(jnp.bfloat16)
    return lax.bitcast_convert_type(wbf.reshape(1024, DW, 2), jnp.int32)


def kernel(bbox, Wx, Wy, Wh, Ww):
    cols = bbox.reshape(N, NQ)
    out = _gather_kernel(cols[:, 0], cols[:, 1], cols[:, 2], cols[:, 3],
                         _prep_table(Wx), _prep_table(Wy),
                         _prep_table(Wh), _prep_table(Ww))
    return out.reshape(B, S, NQ * D)


# speedup vs baseline: 1.8683x; 1.8683x over previous
import functools

import jax
import jax.numpy as jnp
from jax import lax
from jax.experimental import pallas as pl
from jax.experimental.pallas import tpu as pltpu
from jax.experimental.pallas import tpu_sc as plsc

B, S = 64, 512
N = B * S
D = 256
NQ = 4

_info = plsc.get_sparse_core_info()
NC, NS = _info.num_cores, _info.num_subcores
NW = NC * NS
B_PER_W = N // NW
CHUNK = 16
NBUF = 4
N_CHUNKS = B_PER_W // CHUNK
N_GROUPS = N_CHUNKS // NBUF

_mesh = plsc.VectorSubcoreMesh(core_axis_name="c", subcore_axis_name="s")


@functools.partial(
    pl.kernel,
    mesh=_mesh,
    out_type=jax.ShapeDtypeStruct((N, NQ * D), jnp.float32),
    scratch_types=(
        [pltpu.VMEM((B_PER_W,), jnp.int32) for _ in range(NQ)]
        + [pltpu.VMEM((CHUNK, NQ * D), jnp.float32) for _ in range(NBUF)]
        + [pltpu.SemaphoreType.DMA for _ in range(2 * NBUF)]
    ),
)
def _gather_kernel(i0, i1, i2, i3, wx, wy, wh, ww, out, *scratch):
    idx_refs = scratch[:NQ]
    rows = scratch[NQ: NQ + NBUF]
    sem_g = scratch[NQ + NBUF: NQ + 2 * NBUF]
    sem_s = scratch[NQ + 2 * NBUF:]

    tables = (wx, wy, wh, ww)
    idx_hbm = (i0, i1, i3, i2)

    wid = lax.axis_index("s") * NC + lax.axis_index("c")
    base0 = pl.multiple_of(wid * B_PER_W, B_PER_W)

    for q in range(NQ):
        pltpu.sync_copy(idx_hbm[q].at[pl.ds(base0, B_PER_W)], idx_refs[q])

    def group(g, carry):
        goff = pl.multiple_of(g * (NBUF * CHUNK), NBUF * CHUNK)
        for b in range(NBUF):
            off = goff + b * CHUNK

            @pl.when(g > 0)
            def _drain_prev_stores():
                pltpu.make_async_copy(
                    rows[b], out.at[pl.ds(base0, CHUNK)], sem_s[b]).wait()

            for q in range(NQ):
                pltpu.async_copy(
                    tables[q].at[idx_refs[q].at[pl.ds(off, CHUNK)]],
                    rows[b].at[:, pl.ds(q * D, D)], sem_g[b])
        for b in range(NBUF):
            base = base0 + goff + b * CHUNK
            for q in range(NQ):
                pltpu.make_async_copy(
                    tables[q].at[idx_refs[q].at[pl.ds(0, CHUNK)]],
                    rows[b].at[:, pl.ds(q * D, D)], sem_g[b]).wait()
            pltpu.async_copy(rows[b], out.at[pl.ds(base, CHUNK)], sem_s[b])
        return carry

    lax.fori_loop(0, N_GROUPS, group, 0)

    for b in range(NBUF):
        pltpu.make_async_copy(
            rows[b], out.at[pl.ds(base0, CHUNK)], sem_s[b]).wait()


def kernel(bbox, Wx, Wy, Wh, Ww):
    cols = bbox.reshape(N, NQ)
    out = _gather_kernel(cols[:, 0], cols[:, 1], cols[:, 2], cols[:, 3],
                         Wx, Wy, Wh, Ww)
    return out.reshape(B, S, NQ * D)


# speedup vs baseline: 1.8743x; 1.0032x over previous
import functools

import jax
import jax.numpy as jnp
from jax import lax
from jax.experimental import pallas as pl
from jax.experimental.pallas import tpu as pltpu
from jax.experimental.pallas import tpu_sc as plsc

B, S = 64, 512
N = B * S
D = 256
NQ = 4

_info = plsc.get_sparse_core_info()
NC, NS = _info.num_cores, _info.num_subcores
NW = NC * NS
B_PER_W = N // NW
CHUNK = 16
NBUF = 4
N_CHUNKS = B_PER_W // CHUNK
N_GROUPS = N_CHUNKS // NBUF

_mesh = plsc.VectorSubcoreMesh(core_axis_name="c", subcore_axis_name="s")


@functools.partial(
    pl.kernel,
    mesh=_mesh,
    out_type=jax.ShapeDtypeStruct((N, NQ * D), jnp.float32),
    scratch_types=(
        [pltpu.VMEM((B_PER_W,), jnp.int32) for _ in range(NQ)]
        + [pltpu.VMEM((CHUNK, NQ * D), jnp.float32) for _ in range(NBUF)]
        + [pltpu.SemaphoreType.DMA for _ in range(2 * NBUF)]
    ),
)
def _gather_kernel(i0, i1, i2, i3, wx, wy, wh, ww, out, *scratch):
    idx_refs = scratch[:NQ]
    rows = scratch[NQ: NQ + NBUF]
    sem_g = scratch[NQ + NBUF: NQ + 2 * NBUF]
    sem_s = scratch[NQ + 2 * NBUF:]

    tables = (wx, wy, wh, ww)
    idx_hbm = (i0, i1, i3, i2)

    wid = lax.axis_index("s") * NC + lax.axis_index("c")
    base0 = pl.multiple_of(wid * B_PER_W, B_PER_W)

    for q in range(NQ):
        pltpu.async_copy(idx_hbm[q].at[pl.ds(base0, B_PER_W)], idx_refs[q],
                         sem_g[0])
    for q in range(NQ):
        pltpu.make_async_copy(idx_hbm[q].at[pl.ds(base0, B_PER_W)],
                              idx_refs[q], sem_g[0]).wait()

    def group(g, carry):
        goff = pl.multiple_of(g * (NBUF * CHUNK), NBUF * CHUNK)
        for b in range(NBUF):
            off = goff + b * CHUNK

            @pl.when(g > 0)
            def _drain_prev_stores():
                pltpu.make_async_copy(
                    rows[b], out.at[pl.ds(base0, CHUNK)], sem_s[b]).wait()

            for q in range(NQ):
                pltpu.async_copy(
                    tables[q].at[idx_refs[q].at[pl.ds(off, CHUNK)]],
                    rows[b].at[:, pl.ds(q * D, D)], sem_g[b])
        for b in range(NBUF):
            base = base0 + goff + b * CHUNK
            for q in range(NQ):
                pltpu.make_async_copy(
                    tables[q].at[idx_refs[q].at[pl.ds(0, CHUNK)]],
                    rows[b].at[:, pl.ds(q * D, D)], sem_g[b]).wait()
            pltpu.async_copy(rows[b], out.at[pl.ds(base, CHUNK)], sem_s[b])
        return carry

    lax.fori_loop(0, N_GROUPS, group, 0)

    for b in range(NBUF):
        pltpu.make_async_copy(
            rows[b], out.at[pl.ds(base0, CHUNK)], sem_s[b]).wait()


def kernel(bbox, Wx, Wy, Wh, Ww):
    cols = bbox.reshape(N, NQ)
    out = _gather_kernel(cols[:, 0], cols[:, 1], cols[:, 2], cols[:, 3],
                         Wx, Wy, Wh, Ww)
    return out.reshape(B, S, NQ * D)


# speedup vs baseline: 1.8869x; 1.0067x over previous
import functools

import jax
import jax.numpy as jnp
from jax import lax
from jax.experimental import pallas as pl
from jax.experimental.pallas import tpu as pltpu
from jax.experimental.pallas import tpu_sc as plsc

B, S = 64, 512
N = B * S
D = 256
NQ = 4

_info = plsc.get_sparse_core_info()
NC, NS = _info.num_cores, _info.num_subcores
NW = NC * NS
B_PER_W = N // NW
CHUNK = 8
NBUF = 8
N_CHUNKS = B_PER_W // CHUNK
N_GROUPS = N_CHUNKS // NBUF

_mesh = plsc.VectorSubcoreMesh(core_axis_name="c", subcore_axis_name="s")


@functools.partial(
    pl.kernel,
    mesh=_mesh,
    out_type=jax.ShapeDtypeStruct((N, NQ * D), jnp.float32),
    scratch_types=(
        [pltpu.VMEM((B_PER_W,), jnp.int32) for _ in range(NQ)]
        + [pltpu.VMEM((CHUNK, NQ * D), jnp.float32) for _ in range(NBUF)]
        + [pltpu.SemaphoreType.DMA for _ in range(2 * NBUF)]
    ),
)
def _gather_kernel(i0, i1, i2, i3, wx, wy, wh, ww, out, *scratch):
    idx_refs = scratch[:NQ]
    rows = scratch[NQ: NQ + NBUF]
    sem_g = scratch[NQ + NBUF: NQ + 2 * NBUF]
    sem_s = scratch[NQ + 2 * NBUF:]

    tables = (wx, wy, wh, ww)
    idx_hbm = (i0, i1, i3, i2)

    wid = lax.axis_index("s") * NC + lax.axis_index("c")
    base0 = pl.multiple_of(wid * B_PER_W, B_PER_W)

    for q in range(NQ):
        pltpu.async_copy(idx_hbm[q].at[pl.ds(base0, B_PER_W)], idx_refs[q],
                         sem_g[0])
    for q in range(NQ):
        pltpu.make_async_copy(idx_hbm[q].at[pl.ds(base0, B_PER_W)],
                              idx_refs[q], sem_g[0]).wait()

    def group(g, carry):
        goff = pl.multiple_of(g * (NBUF * CHUNK), NBUF * CHUNK)
        for b in range(NBUF):
            off = goff + b * CHUNK

            @pl.when(g > 0)
            def _drain_prev_stores():
                pltpu.make_async_copy(
                    rows[b], out.at[pl.ds(base0, CHUNK)], sem_s[b]).wait()

            for q in range(NQ):
                pltpu.async_copy(
                    tables[q].at[idx_refs[q].at[pl.ds(off, CHUNK)]],
                    rows[b].at[:, pl.ds(q * D, D)], sem_g[b])
        for b in range(NBUF):
            base = base0 + goff + b * CHUNK
            for q in range(NQ):
                pltpu.make_async_copy(
                    tables[q].at[idx_refs[q].at[pl.ds(0, CHUNK)]],
                    rows[b].at[:, pl.ds(q * D, D)], sem_g[b]).wait()
            pltpu.async_copy(rows[b], out.at[pl.ds(base, CHUNK)], sem_s[b])
        return carry

    lax.fori_loop(0, N_GROUPS, group, 0)

    for b in range(NBUF):
        pltpu.make_async_copy(
            rows[b], out.at[pl.ds(base0, CHUNK)], sem_s[b]).wait()


def kernel(bbox, Wx, Wy, Wh, Ww):
    cols = bbox.reshape(N, NQ)
    out = _gather_kernel(cols[:, 0], cols[:, 1], cols[:, 2], cols[:, 3],
                         Wx, Wy, Wh, Ww)
    return out.reshape(B, S, NQ * D)
